# baseline (speedup 1.0000x reference)
import functools

import jax
import jax.numpy as jnp
from jax import lax
from jax.experimental import pallas as pl
from jax.experimental.pallas import tpu as pltpu
from jax.experimental.pallas import tpu_sc as plsc

VOCAB_ = 1000000
DIM_ = 64
HID_ = 16
PAD_ = 0
BATCH_ = 4096
SEQ_ = 200
LINE_ = 128


_BR = 16384
_NBLK = (VOCAB_ + _BR - 1) // _BR
_HALF = _NBLK * _BR // 2


def _mass_body(tlo_ref, thi_ref, w1t_ref, b1_ref, w2_ref, b2_ref,
               o_ref, o2_ref):
    w1t = w1t_ref[...]
    b1 = b1_ref[...]
    w2 = w2_ref[...]
    b2 = b2_ref[0, 0]

    def mass_of(t):
        x = jnp.dot(w1t, t, preferred_element_type=jnp.float32) + b1
        e2 = jnp.exp(2.0 * x)
        h = 1.0 - 2.0 / (e2 + 1.0)
        return jnp.sum(h * w2, axis=0) + b2

    tlo = tlo_ref[...]
    thi = thi_ref[...]
    o_ref[:, 0, :] = mass_of(tlo).reshape(1, _BR)
    o_ref[:, 1, :] = mass_of(thi).reshape(1, _BR)
    o2_ref[:, :DIM_] = tlo.T
    o2_ref[:, DIM_:] = thi.T


def _mass_tc(table_t, W1, b1, W2, b2):
    w1t = W1.T
    b1r = b1.reshape(HID_, 1)
    b2r = b2.reshape(1, 1)
    nh = _NBLK // 2
    out, taug = pl.pallas_call(
        _mass_body,
        grid=(nh,),
        in_specs=[
            pl.BlockSpec((DIM_, _BR), lambda i: (0, i)),
            pl.BlockSpec((DIM_, _BR), lambda i: (0, i + _NBLK // 2)),
            pl.BlockSpec((HID_, DIM_), lambda i: (0, 0)),
            pl.BlockSpec((HID_, 1), lambda i: (0, 0)),
            pl.BlockSpec((HID_, 1), lambda i: (0, 0)),
            pl.BlockSpec((1, 1), lambda i: (0, 0)),
        ],
        out_specs=[
            pl.BlockSpec((1, 2, _BR), lambda i: (i, 0, 0)),
            pl.BlockSpec((_BR, LINE_), lambda i: (i, 0)),
        ],
        out_shape=[
            jax.ShapeDtypeStruct((nh, 2, _BR), jnp.float32),
            jax.ShapeDtypeStruct((_HALF, LINE_), jnp.float32),
        ],
    )(table_t, table_t, w1t, b1r, W2, b2r)
    mass = jnp.transpose(out, (1, 0, 2)).reshape(_NBLK * _BR)
    return mass, taug


_NW = 32
_RPT = BATCH_ // _NW
_SP = 208
_NL = 16
_NB = 2


def _pool_body(tok_hbm, tokh_hbm, mass_hbm, taug_hbm, out_hbm,
               tok_v, tok_h, mb0, mb1, rows0, rows1, out_buf,
               sem_t, sem0, sem1):
    info = plsc.get_sparse_core_info()
    nc = info.num_cores
    wid = lax.axis_index("s") * nc + lax.axis_index("c")
    base_row = wid * _RPT
    rows = (rows0, rows1)
    mbs = (mb0, mb1)
    sems = (sem0, sem1)

    zf16 = jnp.zeros((_NL,), jnp.float32)

    for rbuf in rows:
        for s in range(SEQ_, _SP):
            for j in range(LINE_ // _NL):
                rbuf[s, pl.ds(j * _NL, _NL)] = zf16
    z16 = jnp.zeros((_NL,), jnp.int32)

    def zero_tok(r, _):
        tok_v[r, pl.ds(192, _NL)] = z16
        return _
    lax.fori_loop(0, _RPT, zero_tok, 0)

    pltpu.async_copy(tok_hbm.at[pl.ds(base_row, _RPT), :],
                     tok_v.at[:, pl.ds(0, SEQ_)], sem_t).wait()
    pltpu.async_copy(tokh_hbm.at[pl.ds(base_row, _RPT), :],
                     tok_h.at[:, pl.ds(0, SEQ_)], sem_t).wait()

    def fire_rows(r, slot, mb, sem):
        pltpu.async_copy(taug_hbm.at[tok_h.at[r, pl.ds(0, 112)]],
                         slot.at[pl.ds(0, 112)], sem)
        pltpu.async_copy(taug_hbm.at[tok_h.at[r, pl.ds(112, 88)]],
                         slot.at[pl.ds(112, 88)], sem)
        pltpu.async_copy(mass_hbm.at[tok_v.at[r, pl.ds(0, 112)]],
                         mb.at[pl.ds(0, 112)], sem)
        pltpu.async_copy(mass_hbm.at[tok_v.at[r, pl.ds(112, 88)]],
                         mb.at[pl.ds(112, 88)], sem)

    def wait_rows(r, slot, mb, sem):
        pltpu.make_async_copy(taug_hbm.at[tok_h.at[r, pl.ds(0, 112)]],
                              slot.at[pl.ds(0, 112)], sem).wait()
        pltpu.make_async_copy(taug_hbm.at[tok_h.at[r, pl.ds(112, 88)]],
                              slot.at[pl.ds(112, 88)], sem).wait()
        pltpu.make_async_copy(mass_hbm.at[tok_v.at[r, pl.ds(0, 112)]],
                              mb.at[pl.ds(0, 112)], sem).wait()
        pltpu.make_async_copy(mass_hbm.at[tok_v.at[r, pl.ds(112, 88)]],
                              mb.at[pl.ds(112, 88)], sem).wait()

    for b in range(_NB):
        fire_rows(b, rows[b], mbs[b], sems[b])

    h16 = jnp.full((_NL,), DIM_, jnp.int32)

    def compute_row(r, slot, mb, sem):
        wait_rows(r, slot, mb, sem)

        def acc_body(c, accs):
            a0, a1, a2, a3 = accs
            base = c * _NL
            tk = tok_v[r, pl.ds(base, _NL)]
            m = mb[pl.ds(base, _NL)]
            wv = jnp.where(tk == PAD_, zf16, jnp.exp(m))
            offv = jnp.where(tk < _HALF, z16, h16)
            for k in range(_NL):
                s = base + k
                w = wv[k]
                off = offv[k]
                a0 = a0 + w * slot[s, pl.ds(off, _NL)]
                a1 = a1 + w * slot[s, pl.ds(off + _NL, _NL)]
                a2 = a2 + w * slot[s, pl.ds(off + 2 * _NL, _NL)]
                a3 = a3 + w * slot[s, pl.ds(off + 3 * _NL, _NL)]
            return (a0, a1, a2, a3)

        a0, a1, a2, a3 = lax.fori_loop(0, 13, acc_body,
                                       (zf16, zf16, zf16, zf16))
        out_buf[r, pl.ds(0, _NL)] = a0
        out_buf[r, pl.ds(_NL, _NL)] = a1
        out_buf[r, pl.ds(2 * _NL, _NL)] = a2
        out_buf[r, pl.ds(3 * _NL, _NL)] = a3

    _NG = _RPT // _NB

    def group_body(g, _):
        for b in range(_NB):
            r = g * _NB + b
            slot, mb, sem = rows[b], mbs[b], sems[b]
            compute_row(r, slot, mb, sem)

            @pl.when(r + _NB < _RPT)
            def _fire_next():
                fire_rows(r + _NB, slot, mb, sem)
        return _

    lax.fori_loop(0, _NG, group_body, 0)
    for r in range(_NG * _NB, _RPT):
        b = r % _NB
        compute_row(r, rows[b], mbs[b], sems[b])
    pltpu.sync_copy(out_buf, out_hbm.at[pl.ds(base_row, _RPT)])


def _pool_sc(token_ids, tok_half, mass, taug):
    mesh = plsc.VectorSubcoreMesh(core_axis_name="c", subcore_axis_name="s")
    f = functools.partial(
        pl.kernel,
        out_type=jax.ShapeDtypeStruct((BATCH_, DIM_), jnp.float32),
        mesh=mesh,
        scratch_types=[
            pltpu.VMEM((_RPT, _SP), jnp.int32),
            pltpu.VMEM((_RPT, _SP), jnp.int32),
            pltpu.VMEM((_SP,), jnp.float32),
            pltpu.VMEM((_SP,), jnp.float32),
            pltpu.VMEM((_SP, LINE_), jnp.float32),
            pltpu.VMEM((_SP, LINE_), jnp.float32),
            pltpu.VMEM((_RPT, DIM_), jnp.float32),
            pltpu.SemaphoreType.DMA,
            pltpu.SemaphoreType.DMA,
            pltpu.SemaphoreType.DMA,
        ],
        compiler_params=pltpu.CompilerParams(use_tc_tiling_on_sc=False),
    )(_pool_body)
    return f(token_ids, tok_half, mass, taug)


def _norm_body(x_ref, o_ref):
    x = x_ref[...]
    n = jnp.sqrt(jnp.sum(x * x, axis=1, keepdims=True))
    o_ref[...] = x / jnp.maximum(n, 1e-12)


def _normalize_tc(sv):
    return pl.pallas_call(
        _norm_body,
        out_shape=jax.ShapeDtypeStruct((BATCH_, DIM_), jnp.float32),
    )(sv)


def kernel(token_ids, table, W1, b1, W2, b2):
    mass, taug = _mass_tc(table.T, W1, b1, W2, b2)
    tok_half = jnp.where(token_ids >= _HALF, token_ids - _HALF, token_ids)
    sv = _pool_sc(token_ids, tok_half, mass, taug)
    return _normalize_tc(sv)
